# Initial kernel scaffold; baseline (speedup 1.0000x reference)
import jax
import jax.numpy as jnp
from jax.experimental import pallas as pl


def kernel(users_emb, items_emb, adj_indices, adj_values):
    raise NotImplementedError("write your pallas kernel here")


# speedup vs baseline: 3.6239x; 3.6239x over previous
import jax
import jax.numpy as jnp
from jax import lax
from jax.experimental import pallas as pl
from jax.experimental.pallas import tpu as pltpu
from jax.experimental.pallas import tpu_sc as plsc

N_USERS = 5000
N_ITEMS = 5000
N = N_USERS + N_ITEMS
E = 320000
D = 128
N_LAYERS = 3

NC = 2
NP_ = 2
NB = NC * NP_
W = D // NB
Q = W // 16
NS = 16

EPT = E // NS
CHUNK = 128
NCH = EPT // CHUNK
NCH = (EPT + CHUNK - 1) // CHUNK
EPT_P = NCH * CHUNK

N_P = 10240
ROWS_PT = N_P // NS
ECH = 128
NECH = ROWS_PT // ECH


def _scale_chunk(buf, vals_v, b, j):
    def group_body(g, carry):
        e0 = g * 16
        v16 = vals_v[j, pl.ds(e0, 16)]
        for ee in range(16):
            v = v16[ee]
            for q in range(Q):
                sl = pl.ds(q * 16, 16)
                buf[b, e0 + ee, sl] = buf[b, e0 + ee, sl] * v
        return carry
    lax.fori_loop(0, CHUNK // 16, group_body, 0)


def _gcn_body(table0, cols_h, rows_h, vals_h, out, tabA, tabB,
              cols_v, rows_v, vals_v, buf, zbuf, acc, gsem, ssem):
    c = lax.axis_index("c")
    s = lax.axis_index("s")

    for p in range(NP_):
        pltpu.sync_copy(cols_h.at[c, p, s], cols_v.at[p])
    pltpu.sync_copy(rows_h.at[s], rows_v)
    pltpu.sync_copy(vals_h.at[s], vals_v)

    def zb(i, carry):
        for q in range(Q):
            zbuf[i, pl.ds(q * 16, 16)] = jnp.zeros((16,), jnp.float32)
        return carry
    lax.fori_loop(0, CHUNK, zb, 0)

    base = s * ROWS_PT
    for k in range(NECH):
        pltpu.sync_copy(zbuf, acc.at[pl.ds(base + k * ECH, ECH)])
    for p in range(NP_):
        for k in range(NECH):
            hs = pl.ds((NP_ * c + p) * N_P + base + k * ECH, ECH)
            pltpu.sync_copy(table0.at[hs], buf.at[0])
            pltpu.sync_copy(buf.at[0], out.at[hs])
    plsc.subcore_barrier()

    srcs = [table0, tabA, tabB]
    dsts = [tabA, tabB, None]

    for layer in range(N_LAYERS):
        src = srcs[layer]
        dst = dsts[layer]
        last = layer == N_LAYERS - 1

        for p in range(NP_):
            def chunk_body(j, carry):
                pltpu.sync_copy(src.at[cols_v.at[p, j]], buf.at[0])
                _scale_chunk(buf, vals_v, 0, j)
                pltpu.sync_copy(buf.at[0], acc.at[rows_v.at[j]], add=True)
                return carry
            lax.fori_loop(0, NCH, chunk_body, 0)
            plsc.subcore_barrier()

            for k in range(NECH):
                r0 = base + k * ECH
                sl = pl.ds(r0, ECH)
                hs = pl.ds((NP_ * c + p) * N_P + r0, ECH)
                ba = buf.at[0]
                bs = buf.at[1]
                pltpu.sync_copy(acc.at[sl], ba)
                pltpu.sync_copy(out.at[hs], bs)

                def add_body(e, carry):
                    for q in range(Q):
                        qs = pl.ds(q * 16, 16)
                        t = buf[1, e, qs] + buf[0, e, qs]
                        if last:
                            t = t * 0.25
                        buf[1, e, qs] = t
                    return carry
                lax.fori_loop(0, ECH, add_body, 0, unroll=2)

                pltpu.sync_copy(bs, out.at[hs])
                if not last:
                    pltpu.sync_copy(ba, dst.at[hs])
                pltpu.sync_copy(zbuf, acc.at[sl])
            plsc.subcore_barrier()


@jax.jit
def kernel(users_emb, items_emb, adj_indices, adj_values):
    all_emb = jnp.concatenate([users_emb, items_emb], axis=0)
    all_emb = jnp.pad(all_emb, ((0, N_P - N), (0, 0)))
    table0 = jnp.reshape(
        jnp.transpose(jnp.reshape(all_emb, (N_P, NB, W)), (1, 0, 2)),
        (NB * N_P, W))

    rows = adj_indices[0].reshape(NS, EPT)
    cols = adj_indices[1].reshape(NS, EPT)
    vals = adj_values.reshape(NS, EPT)
    pad = EPT_P - EPT
    rows = jnp.pad(rows, ((0, 0), (0, pad)))
    cols = jnp.pad(cols, ((0, 0), (0, pad)))
    vals = jnp.pad(vals, ((0, 0), (0, pad)))
    cols2 = jnp.stack([cols + b * N_P for b in range(NB)])
    cols2 = cols2.reshape(NC, NP_, NS, NCH, CHUNK)
    rows3 = rows.reshape(NS, NCH, CHUNK)
    vals3 = vals.reshape(NS, NCH, CHUNK)

    mesh = plsc.VectorSubcoreMesh(core_axis_name="c", subcore_axis_name="s")
    f32 = jnp.float32
    run = pl.kernel(
        _gcn_body,
        out_type=(
            jax.ShapeDtypeStruct((NB * N_P, W), f32),
            jax.ShapeDtypeStruct((NB * N_P, W), f32),
            jax.ShapeDtypeStruct((NB * N_P, W), f32),
        ),
        mesh=mesh,
        compiler_params=pltpu.CompilerParams(use_tc_tiling_on_sc=False),
        scratch_types=(
            pltpu.VMEM((NP_, NCH, CHUNK), jnp.int32),
            pltpu.VMEM((NCH, CHUNK), jnp.int32),
            pltpu.VMEM((NCH, CHUNK), f32),
            pltpu.VMEM((4, CHUNK, W), f32),
            pltpu.VMEM((CHUNK, W), f32),
            pltpu.VMEM_SHARED((N_P, W), f32),
            pltpu.SemaphoreType.DMA((4,)),
            pltpu.SemaphoreType.DMA((4,)),
        ),
    )
    out2, _, _ = run(table0, cols2, rows3, vals3)
    out3 = jnp.transpose(jnp.reshape(out2, (NB, N_P, W)), (1, 0, 2))
    return jnp.reshape(out3[:N], (N, D))


# speedup vs baseline: 5.7933x; 1.5986x over previous
import jax
import jax.numpy as jnp
from jax import lax
from jax.experimental import pallas as pl
from jax.experimental.pallas import tpu as pltpu
from jax.experimental.pallas import tpu_sc as plsc

N_USERS = 5000
N_ITEMS = 5000
N = N_USERS + N_ITEMS
E = 320000
D = 128
N_LAYERS = 3

NC = 2
NP_ = 1
NB = NC * NP_
W = D // NB
Q = W // 16
NS = 16

EPT = E // NS
CHUNK = 128
NBUF = 2
NCH = (EPT + CHUNK - 1) // CHUNK
NCH = ((NCH + NBUF - 1) // NBUF) * NBUF
EPT_P = NCH * CHUNK

N_P = 10240
ROWS_PT = N_P // NS
ECH = 128
NECH = ROWS_PT // ECH


def _scale_chunk(buf, vals_v, b, t, i):
    @plsc.parallel_loop(0, CHUNK // 16, unroll=2)
    def group_body(g):
        e0 = g * 16
        v16 = vals_v[t, i, pl.ds(e0, 16)]
        for ee in range(16):
            v = v16[ee]
            for q in range(Q):
                sl = pl.ds(q * 16, 16)
                buf[b, e0 + ee, sl] = buf[b, e0 + ee, sl] * v


def _gcn_body(table0, cols_h, rows_h, vals_h, out, tabA, tabB,
              cols_v, rows_v, vals_v, buf, zbuf, acc, gsems, ssems):
    c = lax.axis_index("c")
    s = lax.axis_index("s")

    for p in range(NP_):
        pltpu.sync_copy(cols_h.at[c, p, s], cols_v.at[p])
    pltpu.sync_copy(rows_h.at[s], rows_v)
    pltpu.sync_copy(vals_h.at[s], vals_v)

    def zb(i, carry):
        for q in range(Q):
            zbuf[i, pl.ds(q * 16, 16)] = jnp.zeros((16,), jnp.float32)
        return carry
    lax.fori_loop(0, CHUNK, zb, 0)

    base = s * ROWS_PT
    for k in range(NECH):
        pltpu.sync_copy(zbuf, acc.at[pl.ds(base + k * ECH, ECH)])
    for p in range(NP_):
        for k in range(NECH):
            hs = pl.ds((NP_ * c + p) * N_P + base + k * ECH, ECH)
            pltpu.sync_copy(table0.at[hs], buf.at[0])
            pltpu.sync_copy(buf.at[0], out.at[hs])
    plsc.subcore_barrier()

    srcs = [table0, tabA, tabB]
    dsts = [tabA, tabB, None]

    for layer in range(N_LAYERS):
        src = srcs[layer]
        dst = dsts[layer]
        last = layer == N_LAYERS - 1

        for p in range(NP_):
            def oct_body(i, carry):
                gds = [
                    pltpu.async_copy(src.at[cols_v.at[p, t, i]],
                                     buf.at[t], gsems[t])
                    for t in range(NBUF)
                ]
                sds = []
                for t in range(NBUF):
                    gds[t].wait()
                    _scale_chunk(buf, vals_v, t, t, i)
                    sds.append(
                        pltpu.async_copy(buf.at[t],
                                         acc.at[rows_v.at[t, i]],
                                         ssems[t], add=True))
                for t in range(NBUF):
                    sds[t].wait()
                return carry
            lax.fori_loop(0, NCH // NBUF, oct_body, 0)
            plsc.subcore_barrier()

            for k in range(NECH):
                r0 = base + k * ECH
                sl = pl.ds(r0, ECH)
                hs = pl.ds((NP_ * c + p) * N_P + r0, ECH)
                ba = buf.at[0]
                bs = buf.at[1]
                pltpu.sync_copy(acc.at[sl], ba)
                pltpu.sync_copy(out.at[hs], bs)

                def add_body(e, carry):
                    for q in range(Q):
                        qs = pl.ds(q * 16, 16)
                        t = buf[1, e, qs] + buf[0, e, qs]
                        if last:
                            t = t * 0.25
                        buf[1, e, qs] = t
                    return carry
                lax.fori_loop(0, ECH, add_body, 0, unroll=2)

                pltpu.sync_copy(bs, out.at[hs])
                if not last:
                    pltpu.sync_copy(ba, dst.at[hs])
                pltpu.sync_copy(zbuf, acc.at[sl])
            plsc.subcore_barrier()


@jax.jit
def kernel(users_emb, items_emb, adj_indices, adj_values):
    all_emb = jnp.concatenate([users_emb, items_emb], axis=0)
    all_emb = jnp.pad(all_emb, ((0, N_P - N), (0, 0)))
    table0 = jnp.reshape(
        jnp.transpose(jnp.reshape(all_emb, (N_P, NB, W)), (1, 0, 2)),
        (NB * N_P, W))

    rows = adj_indices[0].reshape(NS, EPT)
    cols = adj_indices[1].reshape(NS, EPT)
    vals = adj_values.reshape(NS, EPT)
    pad = EPT_P - EPT
    rows = jnp.pad(rows, ((0, 0), (0, pad)))
    cols = jnp.pad(cols, ((0, 0), (0, pad)))
    vals = jnp.pad(vals, ((0, 0), (0, pad)))
    cols2 = jnp.stack([cols + b * N_P for b in range(NB)])
    cols2 = cols2.reshape(NC, NP_, NS, NBUF, NCH // NBUF, CHUNK)
    rows3 = rows.reshape(NS, NBUF, NCH // NBUF, CHUNK)
    vals3 = vals.reshape(NS, NBUF, NCH // NBUF, CHUNK)

    mesh = plsc.VectorSubcoreMesh(core_axis_name="c", subcore_axis_name="s")
    f32 = jnp.float32
    run = pl.kernel(
        _gcn_body,
        out_type=(
            jax.ShapeDtypeStruct((NB * N_P, W), f32),
            jax.ShapeDtypeStruct((NB * N_P, W), f32),
            jax.ShapeDtypeStruct((NB * N_P, W), f32),
        ),
        mesh=mesh,
        compiler_params=pltpu.CompilerParams(use_tc_tiling_on_sc=False),
        scratch_types=(
            pltpu.VMEM((NP_, NBUF, NCH // NBUF, CHUNK), jnp.int32),
            pltpu.VMEM((NBUF, NCH // NBUF, CHUNK), jnp.int32),
            pltpu.VMEM((NBUF, NCH // NBUF, CHUNK), f32),
            pltpu.VMEM((NBUF, CHUNK, W), f32),
            pltpu.VMEM((CHUNK, W), f32),
            pltpu.VMEM_SHARED((N_P, W), f32),
            tuple(pltpu.SemaphoreType.DMA for _ in range(NBUF)),
            tuple(pltpu.SemaphoreType.DMA for _ in range(NBUF)),
        ),
    )
    out2, _, _ = run(table0, cols2, rows3, vals3)
    out3 = jnp.transpose(jnp.reshape(out2, (NB, N_P, W)), (1, 0, 2))
    return jnp.reshape(out3[:N], (N, D))


# speedup vs baseline: 6.3841x; 1.1020x over previous
import jax
import jax.numpy as jnp
from jax import lax
from jax.experimental import pallas as pl
from jax.experimental.pallas import tpu as pltpu
from jax.experimental.pallas import tpu_sc as plsc

N_USERS = 5000
N_ITEMS = 5000
N = N_USERS + N_ITEMS
E = 320000
D = 128
N_LAYERS = 3

NC = 2
NP_ = 1
NB = NC * NP_
W = D // NB
Q = W // 16
NS = 16

EPT = E // NS
CHUNK = 128
NBUF = 2
NCH = (EPT + CHUNK - 1) // CHUNK
NCH = ((NCH + NBUF - 1) // NBUF) * NBUF
EPT_P = NCH * CHUNK

N_P = 10240
ROWS_PT = N_P // NS
ECH = 128
NECH = ROWS_PT // ECH


def _scale_chunk(buf, vals_v, b, t, i):
    @plsc.parallel_loop(0, CHUNK // 16, unroll=2)
    def group_body(g):
        e0 = g * 16
        v16 = vals_v[t, i, pl.ds(e0, 16)]
        for ee in range(16):
            v = v16[ee]
            for q in range(Q):
                sl = pl.ds(q * 16, 16)
                buf[b, e0 + ee, sl] = buf[b, e0 + ee, sl] * v


def _gcn_body(table0, cols_h, rows_h, vals_h, out, tabA, tabB,
              cols_v, rows_v, vals_v, buf, zbuf, acc, gsems, ssems):
    c = lax.axis_index("c")
    s = lax.axis_index("s")

    for p in range(NP_):
        pltpu.sync_copy(cols_h.at[c, p, s], cols_v.at[p])
    pltpu.sync_copy(rows_h.at[s], rows_v)
    pltpu.sync_copy(vals_h.at[s], vals_v)

    def zb(i, carry):
        for q in range(Q):
            zbuf[i, pl.ds(q * 16, 16)] = jnp.zeros((16,), jnp.float32)
        return carry
    lax.fori_loop(0, CHUNK, zb, 0)

    base = s * ROWS_PT
    for k in range(NECH):
        pltpu.sync_copy(zbuf, acc.at[pl.ds(base + k * ECH, ECH)])
    for p in range(NP_):
        for k in range(NECH):
            hs = pl.ds((NP_ * c + p) * N_P + base + k * ECH, ECH)
            pltpu.sync_copy(table0.at[hs], buf.at[0])
            pltpu.sync_copy(buf.at[0], out.at[hs])
    plsc.subcore_barrier()

    srcs = [table0, tabA, tabB]
    dsts = [tabA, tabB, None]

    for layer in range(N_LAYERS):
        src = srcs[layer]
        dst = dsts[layer]
        last = layer == N_LAYERS - 1

        for p in range(NP_):
            NIT = NCH // NBUF
            for t in range(NBUF):
                pltpu.async_copy(src.at[cols_v.at[p, t, 0]], buf.at[t],
                                 gsems[t])

            def oct_body(i, carry):
                sds = []
                for t in range(NBUF):
                    pltpu.make_async_copy(src.at[cols_v.at[p, t, i]],
                                          buf.at[t], gsems[t]).wait()
                    _scale_chunk(buf, vals_v, t, t, i)
                    sds.append(
                        pltpu.async_copy(buf.at[t],
                                         acc.at[rows_v.at[t, i]],
                                         ssems[t], add=True))
                for t in range(NBUF):
                    sds[t].wait()

                    @pl.when(i < NIT - 1)
                    def _():
                        pltpu.async_copy(src.at[cols_v.at[p, t, i + 1]],
                                         buf.at[t], gsems[t])
                return carry
            lax.fori_loop(0, NIT, oct_body, 0)
            plsc.subcore_barrier()

            for k in range(NECH):
                r0 = base + k * ECH
                sl = pl.ds(r0, ECH)
                hs = pl.ds((NP_ * c + p) * N_P + r0, ECH)
                ba = buf.at[0]
                bs = buf.at[1]
                pltpu.sync_copy(acc.at[sl], ba)
                pltpu.sync_copy(out.at[hs], bs)

                def add_body(e, carry):
                    for q in range(Q):
                        qs = pl.ds(q * 16, 16)
                        t = buf[1, e, qs] + buf[0, e, qs]
                        if last:
                            t = t * 0.25
                        buf[1, e, qs] = t
                    return carry
                lax.fori_loop(0, ECH, add_body, 0, unroll=2)

                pltpu.sync_copy(bs, out.at[hs])
                if not last:
                    pltpu.sync_copy(ba, dst.at[hs])
                pltpu.sync_copy(zbuf, acc.at[sl])
            plsc.subcore_barrier()


@jax.jit
def kernel(users_emb, items_emb, adj_indices, adj_values):
    all_emb = jnp.concatenate([users_emb, items_emb], axis=0)
    all_emb = jnp.pad(all_emb, ((0, N_P - N), (0, 0)))
    table0 = jnp.reshape(
        jnp.transpose(jnp.reshape(all_emb, (N_P, NB, W)), (1, 0, 2)),
        (NB * N_P, W))

    rows = adj_indices[0].reshape(NS, EPT)
    cols = adj_indices[1].reshape(NS, EPT)
    vals = adj_values.reshape(NS, EPT)
    pad = EPT_P - EPT
    rows = jnp.pad(rows, ((0, 0), (0, pad)))
    cols = jnp.pad(cols, ((0, 0), (0, pad)))
    vals = jnp.pad(vals, ((0, 0), (0, pad)))
    cols2 = jnp.stack([cols + b * N_P for b in range(NB)])
    cols2 = cols2.reshape(NC, NP_, NS, NBUF, NCH // NBUF, CHUNK)
    rows3 = rows.reshape(NS, NBUF, NCH // NBUF, CHUNK)
    vals3 = vals.reshape(NS, NBUF, NCH // NBUF, CHUNK)

    mesh = plsc.VectorSubcoreMesh(core_axis_name="c", subcore_axis_name="s")
    f32 = jnp.float32
    run = pl.kernel(
        _gcn_body,
        out_type=(
            jax.ShapeDtypeStruct((NB * N_P, W), f32),
            jax.ShapeDtypeStruct((NB * N_P, W), f32),
            jax.ShapeDtypeStruct((NB * N_P, W), f32),
        ),
        mesh=mesh,
        compiler_params=pltpu.CompilerParams(use_tc_tiling_on_sc=False),
        scratch_types=(
            pltpu.VMEM((NP_, NBUF, NCH // NBUF, CHUNK), jnp.int32),
            pltpu.VMEM((NBUF, NCH // NBUF, CHUNK), jnp.int32),
            pltpu.VMEM((NBUF, NCH // NBUF, CHUNK), f32),
            pltpu.VMEM((NBUF, CHUNK, W), f32),
            pltpu.VMEM((CHUNK, W), f32),
            pltpu.VMEM_SHARED((N_P, W), f32),
            tuple(pltpu.SemaphoreType.DMA for _ in range(NBUF)),
            tuple(pltpu.SemaphoreType.DMA for _ in range(NBUF)),
        ),
    )
    out2, _, _ = run(table0, cols2, rows3, vals3)
    out3 = jnp.transpose(jnp.reshape(out2, (NB, N_P, W)), (1, 0, 2))
    return jnp.reshape(out3[:N], (N, D))
